# baseline (speedup 1.0000x reference)
import jax
import jax.numpy as jnp
from jax.experimental import pallas as pl
from jax.experimental.pallas import tpu as pltpu

_BN = 256
_K = 8192
_D = 256
_H = _K // 2


def _c2_kernel(cb_ref, c2_ref):
    cb = cb_ref[...]
    c2_ref[...] = jnp.sum(cb * cb, axis=1)


def _vq_kernel(x_ref, cb_ref, c2_ref, q_ref, ids_ref):
    xb = x_ref[...]
    cb = cb_ref[...]
    x2 = jnp.sum(xb * xb, axis=1, keepdims=True)
    dot = jax.lax.dot_general(
        xb, cb, (((1,), (1,)), ((), ())),
        preferred_element_type=jnp.float32)
    c2 = c2_ref[...][None, :]
    d2 = jnp.maximum((x2 + c2) - 2.0 * dot, 0.0)
    dist = -jnp.sqrt(d2)

    lo, hi = dist[:, :_H], dist[:, _H:]
    i0 = jnp.argmax(lo, axis=1).astype(jnp.int32)
    m0 = jnp.max(lo, axis=1)
    i1 = jnp.argmax(hi, axis=1).astype(jnp.int32)
    m1 = jnp.max(hi, axis=1)
    m0q = m0.astype(jnp.bfloat16).astype(jnp.float32)
    ids = jnp.where(m1 > m0q, i1 + _H, i0)
    ids_ref[...] = ids

    onehot = (ids[:, None]
              == jax.lax.broadcasted_iota(jnp.int32, (_BN, _K), 1))
    oh = onehot.astype(jnp.bfloat16)
    cb_hi = cb.astype(jnp.bfloat16)
    cb_lo = (cb - cb_hi.astype(jnp.float32)).astype(jnp.bfloat16)
    q = jax.lax.dot_general(oh, cb_hi, (((1,), (0,)), ((), ())),
                            preferred_element_type=jnp.float32)
    q += jax.lax.dot_general(oh, cb_lo, (((1,), (0,)), ((), ())),
                             preferred_element_type=jnp.float32)
    q_ref[...] = q


@jax.jit
def kernel(x, CodeBook):
    n = x.shape[0]
    c2 = pl.pallas_call(
        _c2_kernel,
        out_shape=jax.ShapeDtypeStruct((_K,), jnp.float32),
    )(CodeBook)

    grid = (n // _BN,)
    q, ids = pl.pallas_call(
        _vq_kernel,
        grid=grid,
        in_specs=[
            pl.BlockSpec((_BN, _D), lambda i: (i, 0)),
            pl.BlockSpec((_K, _D), lambda i: (0, 0)),
            pl.BlockSpec((_K,), lambda i: (0,)),
        ],
        out_specs=[
            pl.BlockSpec((_BN, _D), lambda i: (i, 0)),
            pl.BlockSpec((_BN,), lambda i: (i,)),
        ],
        out_shape=[
            jax.ShapeDtypeStruct((n, _D), jnp.float32),
            jax.ShapeDtypeStruct((n,), jnp.int32),
        ],
        compiler_params=pltpu.CompilerParams(
            dimension_semantics=("parallel",),
        ),
    )(x, CodeBook, c2)
    return (q, ids)
